# baseline (speedup 1.0000x reference)
import functools

import jax
import jax.numpy as jnp
from jax.experimental import pallas as pl

_NEG = -3.0e38


def _feat2d_kernel(x_ref, w_ref, b_ref, o_ref, *, bb, n):
    p = n * n
    neg = jnp.bfloat16(_NEG)
    xt = x_ref[...].astype(jnp.bfloat16)
    g = jnp.broadcast_to(xt[:, :, None, :], (bb, n, n, xt.shape[-1]))
    g = g.reshape(bb, p, xt.shape[-1])
    pos = jax.lax.broadcasted_iota(jnp.int32, (1, p, 1), 1)
    i_idx = pos // n
    j_idx = pos - i_idx * n
    mask_u = j_idx >= i_idx
    c = jnp.where(mask_u, g, neg)
    rows = [None] * n
    prev = None
    for i in reversed(range(n)):
        cur = c[:, i * n:(i + 1) * n, :]
        if prev is not None:
            cur = jnp.maximum(cur, prev)
        rows[i] = cur
        prev = cur
    c = jnp.concatenate(rows, axis=1)
    m = jnp.where(mask_u, c, jnp.bfloat16(0.0))
    w = w_ref[...].astype(jnp.bfloat16)
    bias = b_ref[...]
    for k in range(bb):
        o_ref[k, :, :] = jax.lax.dot_general(
            m[k], w, (((1,), (1,)), ((), ())),
            preferred_element_type=jnp.float32) + bias


@jax.jit
def kernel(x, W, b):
    B, D, N = x.shape
    P = N * N
    BB = 64
    xt = jnp.swapaxes(x, 1, 2)
    out = pl.pallas_call(
        functools.partial(_feat2d_kernel, bb=BB, n=N),
        grid=(B // BB,),
        in_specs=[
            pl.BlockSpec((BB, N, D), lambda g: (g, 0, 0)),
            pl.BlockSpec((D, D), lambda g: (0, 0)),
            pl.BlockSpec((1, D), lambda g: (0, 0)),
        ],
        out_specs=pl.BlockSpec((BB, P, D), lambda g: (g, 0, 0)),
        out_shape=jax.ShapeDtypeStruct((B, P, D), jnp.float32),
    )(xt, W, b.reshape(1, D))
    return out.reshape(B, N, N, D).transpose(0, 3, 1, 2)


# speedup vs baseline: 1.0194x; 1.0194x over previous
import functools

import jax
import jax.numpy as jnp
from jax.experimental import pallas as pl

_NEG = -3.0e38


def _feat2d_kernel(x_ref, w_ref, b_ref, o_ref, *, bb, n):
    p = n * n
    neg = jnp.bfloat16(_NEG)
    xt = x_ref[...].astype(jnp.bfloat16)
    g = jnp.broadcast_to(xt[:, :, None, :], (bb, n, n, xt.shape[-1]))
    g = g.reshape(bb, p, xt.shape[-1])
    pos = jax.lax.broadcasted_iota(jnp.int32, (1, p, 1), 1)
    i_idx = pos // n
    j_idx = pos - i_idx * n
    mask_u = j_idx >= i_idx
    c = jnp.where(mask_u, g, neg)
    rows = [None] * n
    prev = None
    for i in reversed(range(n)):
        cur = c[:, i * n:(i + 1) * n, :]
        if prev is not None:
            cur = jnp.maximum(cur, prev)
        rows[i] = cur
        prev = cur
    c = jnp.concatenate(rows, axis=1)
    m = jnp.where(mask_u, c, jnp.bfloat16(0.0))
    w = w_ref[...].astype(jnp.bfloat16)
    bias = b_ref[...]
    for k in range(bb):
        o_ref[k, :, :] = jax.lax.dot_general(
            m[k], w, (((1,), (1,)), ((), ())),
            preferred_element_type=jnp.float32) + bias


@jax.jit
def kernel(x, W, b):
    B, D, N = x.shape
    P = N * N
    BB = 32
    xt = jnp.swapaxes(x, 1, 2)
    out = pl.pallas_call(
        functools.partial(_feat2d_kernel, bb=BB, n=N),
        grid=(B // BB,),
        in_specs=[
            pl.BlockSpec((BB, N, D), lambda g: (g, 0, 0)),
            pl.BlockSpec((D, D), lambda g: (0, 0)),
            pl.BlockSpec((1, D), lambda g: (0, 0)),
        ],
        out_specs=pl.BlockSpec((BB, P, D), lambda g: (g, 0, 0)),
        out_shape=jax.ShapeDtypeStruct((B, P, D), jnp.float32),
    )(xt, W, b.reshape(1, D))
    return out.reshape(B, N, N, D).transpose(0, 3, 1, 2)
